# baseline (speedup 1.0000x reference)
import functools

import jax
import jax.numpy as jnp
from jax import lax
from jax.experimental import pallas as pl
from jax.experimental.pallas import tpu as pltpu
from jax.experimental.pallas import tpu_sc as plsc

NC = 2
NS = 16
B = 128

_MESH = plsc.VectorSubcoreMesh(
    core_axis_name="c", subcore_axis_name="s", num_cores=NC, num_subcores=NS
)


def _make_deg_kernel(CH, NACC, RPT):
    @functools.partial(
        pl.kernel,
        out_type=jax.ShapeDtypeStruct((NC * NACC,), jnp.float32),
        mesh=_MESH,
        scratch_types=[
            pltpu.VMEM((CH, B), jnp.int32),
            pltpu.VMEM((B,), jnp.float32),
            pltpu.VMEM_SHARED((NACC,), jnp.float32),
        ],
        compiler_params=pltpu.CompilerParams(use_tc_tiling_on_sc=False),
    )
    def deg_kernel(dst_hbm, zeros_hbm, out_hbm, dstv, onesv, acc):
        c = lax.axis_index("c")
        s = lax.axis_index("s")
        w = c * NS + s
        pltpu.sync_copy(dst_hbm.at[w], dstv)
        for i in range(B // 16):
            onesv[pl.ds(i * 16, 16)] = jnp.ones((16,), jnp.float32)
        pltpu.sync_copy(zeros_hbm, acc.at[pl.ds(s * RPT, RPT)])
        plsc.subcore_barrier()

        def body(ch, carry):
            pltpu.sync_copy(onesv, acc.at[dstv.at[ch]], add=True)
            return carry

        lax.fori_loop(0, CH, body, 0)
        plsc.subcore_barrier()
        pltpu.sync_copy(acc.at[pl.ds(s * RPT, RPT)],
                        out_hbm.at[pl.ds(c * NACC + s * RPT, RPT)])

    return deg_kernel


def _make_agg_kernel(N, D, CH, NACC, RPT):
    @functools.partial(
        pl.kernel,
        out_type=jax.ShapeDtypeStruct((NACC, 128), jnp.float32),
        mesh=_MESH,
        scratch_types=[
            pltpu.VMEM((CH, B), jnp.int32),
            pltpu.VMEM((CH, B), jnp.int32),
            pltpu.VMEM((4, B, D), jnp.float32),
            pltpu.VMEM_SHARED((NACC, D), jnp.float32),
            pltpu.SemaphoreType.DMA,
            pltpu.SemaphoreType.DMA,
            pltpu.SemaphoreType.DMA,
            pltpu.SemaphoreType.DMA,
        ],
        compiler_params=pltpu.CompilerParams(use_tc_tiling_on_sc=False),
    )
    def agg_kernel(g_hbm, src_hbm, dst_hbm, zeros_hbm, out_hbm,
                   srcv, dstv, rows, acc, sem0, sem1, sem2, sem3):
        c = lax.axis_index("c")
        s = lax.axis_index("s")
        w = c * NS + s
        sems = (sem0, sem1, sem2, sem3)
        pltpu.sync_copy(src_hbm.at[w], srcv)
        pltpu.sync_copy(dst_hbm.at[w], dstv)
        pltpu.sync_copy(zeros_hbm, acc.at[pl.ds(s * RPT, RPT)])
        plsc.subcore_barrier()

        def start(ch, b):
            pltpu.async_copy(g_hbm.at[srcv.at[ch]], rows.at[b], sems[b])

        def wait_g(ch, b):
            pltpu.make_async_copy(g_hbm.at[srcv.at[ch]], rows.at[b],
                                  sems[b]).wait()

        start(0, 0)
        start(1, 1)
        start(2, 2)

        def group(gi, carry):
            ch0 = 4 * gi
            for b in range(4):
                ch = ch0 + b

                @pl.when(ch + 3 < CH)
                def _():
                    start(ch + 3, (b + 3) % 4)

                wait_g(ch, b)
                pltpu.sync_copy(rows.at[b], acc.at[dstv.at[ch]], add=True)
            return carry

        lax.fori_loop(0, CH // 4, group, 0)
        plsc.subcore_barrier()
        pltpu.sync_copy(acc.at[pl.ds(s * RPT, RPT)],
                        out_hbm.at[pl.ds(s * RPT, RPT), pl.ds(c * D, D)])

    return agg_kernel


_RB = 1024


def _tc_layer1(xp, W1, degp):
    NP, F = xp.shape
    H = W1.shape[1]

    def body(x_ref, w_ref, h_ref):
        h_ref[...] = jnp.dot(x_ref[...], w_ref[...],
                             preferred_element_type=jnp.float32)

    h = pl.pallas_call(
        body,
        grid=(NP // _RB,),
        in_specs=[
            pl.BlockSpec((_RB, F), lambda i: (i, 0)),
            pl.BlockSpec((F, H), lambda i: (0, 0)),
        ],
        out_specs=pl.BlockSpec((_RB, H), lambda i: (i, 0)),
        out_shape=jax.ShapeDtypeStruct((NP, H), jnp.float32),
    )(xp, W1)

    def body2(h_ref, d_ref, g_ref, dis_ref, r_ref):
        deg = (d_ref[0] + d_ref[1] + 1.0)[:, None]
        dis = lax.rsqrt(deg)
        g_ref[...] = h_ref[...] * dis
        dis_ref[...] = dis[:, 0]
        r_ref[...] = (1.0 / deg)[:, 0]

    g, dis, r = pl.pallas_call(
        body2,
        grid=(NP // _RB,),
        in_specs=[
            pl.BlockSpec((_RB, H), lambda i: (i, 0)),
            pl.BlockSpec((NC, _RB), lambda i: (0, i)),
        ],
        out_specs=[
            pl.BlockSpec((_RB, H), lambda i: (i, 0)),
            pl.BlockSpec((_RB,), lambda i: (i,)),
            pl.BlockSpec((_RB,), lambda i: (i,)),
        ],
        out_shape=[
            jax.ShapeDtypeStruct((NP, H), jnp.float32),
            jax.ShapeDtypeStruct((NP,), jnp.float32),
            jax.ShapeDtypeStruct((NP,), jnp.float32),
        ],
    )(h, degp)
    return h, g, dis, r


def _tc_layer2(aggi, h, dis, r, b1row, W2p, H):
    NP = h.shape[0]
    C2 = W2p.shape[1]

    def body(a_ref, h_ref, dis_ref, r_ref, b_ref, w_ref, h2_ref, g2_ref):
        dis = dis_ref[...][:, None]
        agg = a_ref[:, :H] + a_ref[:, H:2 * H]
        y1 = jnp.maximum(
            agg * dis + h_ref[...] * r_ref[...][:, None] + b_ref[...], 0.0)
        h2 = jnp.dot(y1, w_ref[...], preferred_element_type=jnp.float32)
        h2_ref[...] = h2
        g2_ref[...] = h2 * dis

    return pl.pallas_call(
        body,
        grid=(NP // _RB,),
        in_specs=[
            pl.BlockSpec((_RB, 128), lambda i: (i, 0)),
            pl.BlockSpec((_RB, H), lambda i: (i, 0)),
            pl.BlockSpec((_RB,), lambda i: (i,)),
            pl.BlockSpec((_RB,), lambda i: (i,)),
            pl.BlockSpec((1, H), lambda i: (0, 0)),
            pl.BlockSpec((H, C2), lambda i: (0, 0)),
        ],
        out_specs=[
            pl.BlockSpec((_RB, C2), lambda i: (i, 0)),
            pl.BlockSpec((_RB, C2), lambda i: (i, 0)),
        ],
        out_shape=[
            jax.ShapeDtypeStruct((NP, C2), jnp.float32),
            jax.ShapeDtypeStruct((NP, C2), jnp.float32),
        ],
    )(aggi, h, dis, r, b1row, W2p)


def _tc_final(agg2i, h2, dis, r, b2row, C, C2):
    NP = h2.shape[0]

    def body(a_ref, h2_ref, dis_ref, r_ref, b_ref, o_ref):
        agg = a_ref[:, :C2] + a_ref[:, C2:2 * C2]
        val = jnp.maximum(
            agg * dis_ref[...][:, None]
            + h2_ref[...] * r_ref[...][:, None] + b_ref[...], 0.0)
        o_ref[...] = val[:, :C]

    return pl.pallas_call(
        body,
        grid=(NP // _RB,),
        in_specs=[
            pl.BlockSpec((_RB, 128), lambda i: (i, 0)),
            pl.BlockSpec((_RB, C2), lambda i: (i, 0)),
            pl.BlockSpec((_RB,), lambda i: (i,)),
            pl.BlockSpec((_RB,), lambda i: (i,)),
            pl.BlockSpec((1, C2), lambda i: (0, 0)),
        ],
        out_specs=pl.BlockSpec((_RB, C), lambda i: (i, 0)),
        out_shape=jax.ShapeDtypeStruct((NP, C), jnp.float32),
    )(agg2i, h2, dis, r, b2row)


def kernel(x, edge_index, W1, b1, W2, b2):
    N, F = x.shape
    E = edge_index.shape[1]
    H = W1.shape[1]
    C = W2.shape[1]
    C2 = ((C + 15) // 16) * 16

    NW = NC * NS
    CH = -(-E // (NW * B))
    CH = ((CH + 3) // 4) * 4
    EP = NW * CH * B
    RPT = -(-(N + 1) // NS)
    RPT = ((RPT + 127) // 128) * 128
    NACC = NS * RPT

    src = edge_index[0]
    dst = edge_index[1]
    pad = EP - E
    pad_src = jnp.arange(pad, dtype=src.dtype) % N
    pad_dst = N + jnp.arange(pad, dtype=dst.dtype) % (NACC - N)
    srcp = jnp.concatenate([src, pad_src]).reshape(NW, CH, B)
    dstp = jnp.concatenate([dst, pad_dst]).reshape(NW, CH, B)
    srcp = srcp.astype(jnp.int32)
    dstp = dstp.astype(jnp.int32)

    zeros1 = jnp.zeros((RPT,), jnp.float32)
    zerosH = jnp.zeros((RPT, H), jnp.float32)
    zerosC = jnp.zeros((RPT, C2), jnp.float32)

    deg_p = _make_deg_kernel(CH, NACC, RPT)(dstp, zeros1).reshape(NC, NACC)

    xp = jnp.pad(x, ((0, NACC - N), (0, 0)))

    h, g, dis, r = _tc_layer1(xp, W1, deg_p)

    aggi = _make_agg_kernel(N, H, CH, NACC, RPT)(g, srcp, dstp, zerosH)

    W2p = jnp.pad(W2, ((0, 0), (0, C2 - C)))
    b1row = b1[None, :]
    b2row = jnp.pad(b2, (0, C2 - C))[None, :]

    h2, g2 = _tc_layer2(aggi, h, dis, r, b1row, W2p, H)

    agg2i = _make_agg_kernel(N, C2, CH, NACC, RPT)(g2, srcp, dstp, zerosC)

    return _tc_final(agg2i, h2, dis, r, b2row, C, C2)[:N]


# speedup vs baseline: 1.0005x; 1.0005x over previous
import functools

import jax
import jax.numpy as jnp
from jax import lax
from jax.experimental import pallas as pl
from jax.experimental.pallas import tpu as pltpu
from jax.experimental.pallas import tpu_sc as plsc

NC = 2
NS = 16
B = 128

_MESH = plsc.VectorSubcoreMesh(
    core_axis_name="c", subcore_axis_name="s", num_cores=NC, num_subcores=NS
)


def _make_deg_kernel(CH, NACC, RPT):
    @functools.partial(
        pl.kernel,
        out_type=jax.ShapeDtypeStruct((NC * NACC,), jnp.float32),
        mesh=_MESH,
        scratch_types=[
            pltpu.VMEM((CH, B), jnp.int32),
            pltpu.VMEM((B,), jnp.float32),
            pltpu.VMEM_SHARED((NACC,), jnp.float32),
        ],
        compiler_params=pltpu.CompilerParams(use_tc_tiling_on_sc=False),
    )
    def deg_kernel(dst_hbm, zeros_hbm, out_hbm, dstv, onesv, acc):
        c = lax.axis_index("c")
        s = lax.axis_index("s")
        w = c * NS + s
        pltpu.sync_copy(dst_hbm.at[w], dstv)
        for i in range(B // 16):
            onesv[pl.ds(i * 16, 16)] = jnp.ones((16,), jnp.float32)
        pltpu.sync_copy(zeros_hbm, acc.at[pl.ds(s * RPT, RPT)])
        plsc.subcore_barrier()

        def body(ch, carry):
            pltpu.sync_copy(onesv, acc.at[dstv.at[ch]], add=True)
            return carry

        lax.fori_loop(0, CH, body, 0)
        plsc.subcore_barrier()
        pltpu.sync_copy(acc.at[pl.ds(s * RPT, RPT)],
                        out_hbm.at[pl.ds(c * NACC + s * RPT, RPT)])

    return deg_kernel


def _make_agg_kernel(N, D, CH, NACC, RPT):
    @functools.partial(
        pl.kernel,
        out_type=jax.ShapeDtypeStruct((NACC, 128), jnp.float32),
        mesh=_MESH,
        scratch_types=[
            pltpu.VMEM((CH, B), jnp.int32),
            pltpu.VMEM((CH, B), jnp.int32),
            pltpu.VMEM((4, B, D), jnp.float32),
            pltpu.VMEM_SHARED((NACC, D), jnp.float32),
            pltpu.SemaphoreType.DMA,
            pltpu.SemaphoreType.DMA,
            pltpu.SemaphoreType.DMA,
            pltpu.SemaphoreType.DMA,
        ],
        compiler_params=pltpu.CompilerParams(use_tc_tiling_on_sc=False),
    )
    def agg_kernel(g_hbm, src_hbm, dst_hbm, zeros_hbm, out_hbm,
                   srcv, dstv, rows, acc, sem0, sem1, sem2, sem3):
        c = lax.axis_index("c")
        s = lax.axis_index("s")
        w = c * NS + s
        sems = (sem0, sem1, sem2, sem3)
        pltpu.sync_copy(src_hbm.at[w], srcv)
        pltpu.sync_copy(dst_hbm.at[w], dstv)
        pltpu.sync_copy(zeros_hbm, acc.at[pl.ds(s * RPT, RPT)])
        plsc.subcore_barrier()

        def start(ch, b):
            pltpu.async_copy(g_hbm.at[srcv.at[ch]], rows.at[b], sems[b])

        def wait_g(ch, b):
            pltpu.make_async_copy(g_hbm.at[srcv.at[ch]], rows.at[b],
                                  sems[b]).wait()

        start(0, 0)
        start(1, 1)
        start(2, 2)

        def group(gi, carry):
            ch0 = 4 * gi
            for b in range(4):
                ch = ch0 + b

                @pl.when(ch + 3 < CH)
                def _():
                    start(ch + 3, (b + 3) % 4)

                wait_g(ch, b)
                pltpu.sync_copy(rows.at[b], acc.at[dstv.at[ch]], add=True)
            return carry

        lax.fori_loop(0, CH // 4, group, 0)
        plsc.subcore_barrier()
        pltpu.sync_copy(acc.at[pl.ds(s * RPT, RPT)],
                        out_hbm.at[pl.ds(s * RPT, RPT), pl.ds(c * D, D)])

    return agg_kernel


_RB = 1024


def _tc_layer1(xp, W1, degp):
    NP, F = xp.shape
    H = W1.shape[1]

    def body(x_ref, w_ref, d_ref, h_ref, g_ref, dis_ref, r_ref):
        deg = (d_ref[0] + d_ref[1] + 1.0)[:, None]
        dis = lax.rsqrt(deg)
        r = 1.0 / deg
        h = jnp.dot(x_ref[...], w_ref[...], preferred_element_type=jnp.float32)
        h_ref[...] = h
        g_ref[...] = h * dis
        dis_ref[...] = dis[:, 0]
        r_ref[...] = r[:, 0]

    return pl.pallas_call(
        body,
        grid=(NP // _RB,),
        in_specs=[
            pl.BlockSpec((_RB, F), lambda i: (i, 0)),
            pl.BlockSpec((F, H), lambda i: (0, 0)),
            pl.BlockSpec((NC, _RB), lambda i: (0, i)),
        ],
        out_specs=[
            pl.BlockSpec((_RB, H), lambda i: (i, 0)),
            pl.BlockSpec((_RB, H), lambda i: (i, 0)),
            pl.BlockSpec((_RB,), lambda i: (i,)),
            pl.BlockSpec((_RB,), lambda i: (i,)),
        ],
        out_shape=[
            jax.ShapeDtypeStruct((NP, H), jnp.float32),
            jax.ShapeDtypeStruct((NP, H), jnp.float32),
            jax.ShapeDtypeStruct((NP,), jnp.float32),
            jax.ShapeDtypeStruct((NP,), jnp.float32),
        ],
    )(xp, W1, degp)


def _tc_layer2(aggi, h, dis, r, b1row, W2p, H):
    NP = h.shape[0]
    C2 = W2p.shape[1]

    def body(a_ref, h_ref, dis_ref, r_ref, b_ref, w_ref, h2_ref, g2_ref):
        dis = dis_ref[...][:, None]
        agg = a_ref[:, :H] + a_ref[:, H:2 * H]
        y1 = jnp.maximum(
            agg * dis + h_ref[...] * r_ref[...][:, None] + b_ref[...], 0.0)
        h2 = jnp.dot(y1, w_ref[...], preferred_element_type=jnp.float32)
        h2_ref[...] = h2
        g2_ref[...] = h2 * dis

    return pl.pallas_call(
        body,
        grid=(NP // _RB,),
        in_specs=[
            pl.BlockSpec((_RB, 128), lambda i: (i, 0)),
            pl.BlockSpec((_RB, H), lambda i: (i, 0)),
            pl.BlockSpec((_RB,), lambda i: (i,)),
            pl.BlockSpec((_RB,), lambda i: (i,)),
            pl.BlockSpec((1, H), lambda i: (0, 0)),
            pl.BlockSpec((H, C2), lambda i: (0, 0)),
        ],
        out_specs=[
            pl.BlockSpec((_RB, C2), lambda i: (i, 0)),
            pl.BlockSpec((_RB, C2), lambda i: (i, 0)),
        ],
        out_shape=[
            jax.ShapeDtypeStruct((NP, C2), jnp.float32),
            jax.ShapeDtypeStruct((NP, C2), jnp.float32),
        ],
    )(aggi, h, dis, r, b1row, W2p)


def _tc_final(agg2i, h2, dis, r, b2row, C, C2):
    NP = h2.shape[0]

    def body(a_ref, h2_ref, dis_ref, r_ref, b_ref, o_ref):
        agg = a_ref[:, :C2] + a_ref[:, C2:2 * C2]
        val = jnp.maximum(
            agg * dis_ref[...][:, None]
            + h2_ref[...] * r_ref[...][:, None] + b_ref[...], 0.0)
        o_ref[...] = val[:, :C]

    return pl.pallas_call(
        body,
        grid=(NP // _RB,),
        in_specs=[
            pl.BlockSpec((_RB, 128), lambda i: (i, 0)),
            pl.BlockSpec((_RB, C2), lambda i: (i, 0)),
            pl.BlockSpec((_RB,), lambda i: (i,)),
            pl.BlockSpec((_RB,), lambda i: (i,)),
            pl.BlockSpec((1, C2), lambda i: (0, 0)),
        ],
        out_specs=pl.BlockSpec((_RB, C), lambda i: (i, 0)),
        out_shape=jax.ShapeDtypeStruct((NP, C), jnp.float32),
    )(agg2i, h2, dis, r, b2row)


def kernel(x, edge_index, W1, b1, W2, b2):
    N, F = x.shape
    E = edge_index.shape[1]
    H = W1.shape[1]
    C = W2.shape[1]
    C2 = ((C + 15) // 16) * 16

    NW = NC * NS
    CH = -(-E // (NW * B))
    CH = ((CH + 3) // 4) * 4
    EP = NW * CH * B
    RPT = -(-(N + 1) // NS)
    RPT = ((RPT + 127) // 128) * 128
    NACC = NS * RPT

    src = edge_index[0]
    dst = edge_index[1]
    pad = EP - E
    pad_src = jnp.arange(pad, dtype=src.dtype) % N
    pad_dst = N + jnp.arange(pad, dtype=dst.dtype) % (NACC - N)
    srcp = jnp.concatenate([src, pad_src]).reshape(NW, CH, B)
    dstp = jnp.concatenate([dst, pad_dst]).reshape(NW, CH, B)
    srcp = srcp.astype(jnp.int32)
    dstp = dstp.astype(jnp.int32)

    zeros1 = jnp.zeros((RPT,), jnp.float32)
    zerosH = jnp.zeros((RPT, H), jnp.float32)
    zerosC = jnp.zeros((RPT, C2), jnp.float32)

    deg_p = _make_deg_kernel(CH, NACC, RPT)(dstp, zeros1).reshape(NC, NACC)

    xp = jnp.pad(x, ((0, NACC - N), (0, 0)))

    h, g, dis, r = _tc_layer1(xp, W1, deg_p)

    aggi = _make_agg_kernel(N, H, CH, NACC, RPT)(g, srcp, dstp, zerosH)

    W2p = jnp.pad(W2, ((0, 0), (0, C2 - C)))
    b1row = b1[None, :]
    b2row = jnp.pad(b2, (0, C2 - C))[None, :]

    h2, g2 = _tc_layer2(aggi, h, dis, r, b1row, W2p, H)

    agg2i = _make_agg_kernel(N, C2, CH, NACC, RPT)(g2, srcp, dstp, zerosC)

    return _tc_final(agg2i, h2, dis, r, b2row, C, C2)[:N]


# speedup vs baseline: 1.0245x; 1.0240x over previous
import functools

import jax
import jax.numpy as jnp
from jax import lax
from jax.experimental import pallas as pl
from jax.experimental.pallas import tpu as pltpu
from jax.experimental.pallas import tpu_sc as plsc

NC = 2
NS = 16
B = 128

_MESH = plsc.VectorSubcoreMesh(
    core_axis_name="c", subcore_axis_name="s", num_cores=NC, num_subcores=NS
)


def _make_deg_kernel(CH, NACC, RPT):
    @functools.partial(
        pl.kernel,
        out_type=jax.ShapeDtypeStruct((NC * NACC,), jnp.float32),
        mesh=_MESH,
        scratch_types=[
            pltpu.VMEM((CH, B), jnp.int32),
            pltpu.VMEM((B,), jnp.float32),
            pltpu.VMEM_SHARED((NACC,), jnp.float32),
            pltpu.SemaphoreType.DMA,
        ],
        compiler_params=pltpu.CompilerParams(use_tc_tiling_on_sc=False),
    )
    def deg_kernel(dst_hbm, zeros_hbm, out_hbm, dstv, onesv, acc, sem):
        c = lax.axis_index("c")
        s = lax.axis_index("s")
        w = c * NS + s
        pltpu.sync_copy(dst_hbm.at[w], dstv)
        for i in range(B // 16):
            onesv[pl.ds(i * 16, 16)] = jnp.ones((16,), jnp.float32)
        pltpu.sync_copy(zeros_hbm, acc.at[pl.ds(s * RPT, RPT)])
        plsc.subcore_barrier()

        def group(gi, carry):
            for b in range(8):
                pltpu.make_async_copy(
                    onesv, acc.at[dstv.at[8 * gi + b]], sem).start(add=True)
            for b in range(8):
                pltpu.make_async_copy(
                    onesv, acc.at[dstv.at[8 * gi + b]], sem).wait()
            return carry

        lax.fori_loop(0, CH // 8, group, 0)
        plsc.subcore_barrier()
        pltpu.sync_copy(acc.at[pl.ds(s * RPT, RPT)],
                        out_hbm.at[pl.ds(c * NACC + s * RPT, RPT)])

    return deg_kernel


def _make_agg_kernel(N, D, CH, NACC, RPT):
    @functools.partial(
        pl.kernel,
        out_type=jax.ShapeDtypeStruct((NACC, 128), jnp.float32),
        mesh=_MESH,
        scratch_types=[
            pltpu.VMEM((CH, B), jnp.int32),
            pltpu.VMEM((CH, B), jnp.int32),
            pltpu.VMEM((4, B, D), jnp.float32),
            pltpu.VMEM_SHARED((NACC, D), jnp.float32),
            pltpu.SemaphoreType.DMA,
            pltpu.SemaphoreType.DMA,
            pltpu.SemaphoreType.DMA,
            pltpu.SemaphoreType.DMA,
        ],
        compiler_params=pltpu.CompilerParams(use_tc_tiling_on_sc=False),
    )
    def agg_kernel(g_hbm, src_hbm, dst_hbm, zeros_hbm, out_hbm,
                   srcv, dstv, rows, acc, sem0, sem1, sem2, sem3):
        c = lax.axis_index("c")
        s = lax.axis_index("s")
        w = c * NS + s
        sems = (sem0, sem1, sem2, sem3)
        pltpu.sync_copy(src_hbm.at[w], srcv)
        pltpu.sync_copy(dst_hbm.at[w], dstv)
        pltpu.sync_copy(zeros_hbm, acc.at[pl.ds(s * RPT, RPT)])
        plsc.subcore_barrier()

        def start(ch, b):
            pltpu.async_copy(g_hbm.at[srcv.at[ch]], rows.at[b], sems[b])

        def wait_g(ch, b):
            pltpu.make_async_copy(g_hbm.at[srcv.at[ch]], rows.at[b],
                                  sems[b]).wait()

        start(0, 0)
        start(1, 1)
        start(2, 2)

        def group(gi, carry):
            ch0 = 4 * gi
            for b in range(4):
                ch = ch0 + b

                @pl.when(ch + 3 < CH)
                def _():
                    start(ch + 3, (b + 3) % 4)

                wait_g(ch, b)
                pltpu.sync_copy(rows.at[b], acc.at[dstv.at[ch]], add=True)
            return carry

        lax.fori_loop(0, CH // 4, group, 0)
        plsc.subcore_barrier()
        pltpu.sync_copy(acc.at[pl.ds(s * RPT, RPT)],
                        out_hbm.at[pl.ds(s * RPT, RPT), pl.ds(c * D, D)])

    return agg_kernel


_RB = 1024


def _tc_layer1(xp, W1, degp):
    NP, F = xp.shape
    H = W1.shape[1]

    def body(x_ref, w_ref, d_ref, h_ref, g_ref, dis_ref, r_ref):
        deg = (d_ref[0] + d_ref[1] + 1.0)[:, None]
        dis = lax.rsqrt(deg)
        r = 1.0 / deg
        h = jnp.dot(x_ref[...], w_ref[...], preferred_element_type=jnp.float32)
        h_ref[...] = h
        g_ref[...] = h * dis
        dis_ref[...] = dis[:, 0]
        r_ref[...] = r[:, 0]

    return pl.pallas_call(
        body,
        grid=(NP // _RB,),
        in_specs=[
            pl.BlockSpec((_RB, F), lambda i: (i, 0)),
            pl.BlockSpec((F, H), lambda i: (0, 0)),
            pl.BlockSpec((NC, _RB), lambda i: (0, i)),
        ],
        out_specs=[
            pl.BlockSpec((_RB, H), lambda i: (i, 0)),
            pl.BlockSpec((_RB, H), lambda i: (i, 0)),
            pl.BlockSpec((_RB,), lambda i: (i,)),
            pl.BlockSpec((_RB,), lambda i: (i,)),
        ],
        out_shape=[
            jax.ShapeDtypeStruct((NP, H), jnp.float32),
            jax.ShapeDtypeStruct((NP, H), jnp.float32),
            jax.ShapeDtypeStruct((NP,), jnp.float32),
            jax.ShapeDtypeStruct((NP,), jnp.float32),
        ],
    )(xp, W1, degp)


def _tc_layer2(aggi, h, dis, r, b1row, W2p, H):
    NP = h.shape[0]
    C2 = W2p.shape[1]

    def body(a_ref, h_ref, dis_ref, r_ref, b_ref, w_ref, h2_ref, g2_ref):
        dis = dis_ref[...][:, None]
        agg = a_ref[:, :H] + a_ref[:, H:2 * H]
        y1 = jnp.maximum(
            agg * dis + h_ref[...] * r_ref[...][:, None] + b_ref[...], 0.0)
        h2 = jnp.dot(y1, w_ref[...], preferred_element_type=jnp.float32)
        h2_ref[...] = h2
        g2_ref[...] = h2 * dis

    return pl.pallas_call(
        body,
        grid=(NP // _RB,),
        in_specs=[
            pl.BlockSpec((_RB, 128), lambda i: (i, 0)),
            pl.BlockSpec((_RB, H), lambda i: (i, 0)),
            pl.BlockSpec((_RB,), lambda i: (i,)),
            pl.BlockSpec((_RB,), lambda i: (i,)),
            pl.BlockSpec((1, H), lambda i: (0, 0)),
            pl.BlockSpec((H, C2), lambda i: (0, 0)),
        ],
        out_specs=[
            pl.BlockSpec((_RB, C2), lambda i: (i, 0)),
            pl.BlockSpec((_RB, C2), lambda i: (i, 0)),
        ],
        out_shape=[
            jax.ShapeDtypeStruct((NP, C2), jnp.float32),
            jax.ShapeDtypeStruct((NP, C2), jnp.float32),
        ],
    )(aggi, h, dis, r, b1row, W2p)


def _tc_final(agg2i, h2, dis, r, b2row, C, C2):
    NP = h2.shape[0]

    def body(a_ref, h2_ref, dis_ref, r_ref, b_ref, o_ref):
        agg = a_ref[:, :C2] + a_ref[:, C2:2 * C2]
        val = jnp.maximum(
            agg * dis_ref[...][:, None]
            + h2_ref[...] * r_ref[...][:, None] + b_ref[...], 0.0)
        o_ref[...] = val[:, :C]

    return pl.pallas_call(
        body,
        grid=(NP // _RB,),
        in_specs=[
            pl.BlockSpec((_RB, 128), lambda i: (i, 0)),
            pl.BlockSpec((_RB, C2), lambda i: (i, 0)),
            pl.BlockSpec((_RB,), lambda i: (i,)),
            pl.BlockSpec((_RB,), lambda i: (i,)),
            pl.BlockSpec((1, C2), lambda i: (0, 0)),
        ],
        out_specs=pl.BlockSpec((_RB, C), lambda i: (i, 0)),
        out_shape=jax.ShapeDtypeStruct((NP, C), jnp.float32),
    )(agg2i, h2, dis, r, b2row)


def kernel(x, edge_index, W1, b1, W2, b2):
    N, F = x.shape
    E = edge_index.shape[1]
    H = W1.shape[1]
    C = W2.shape[1]
    C2 = ((C + 15) // 16) * 16

    NW = NC * NS
    CH = -(-E // (NW * B))
    CH = ((CH + 7) // 8) * 8
    EP = NW * CH * B
    RPT = -(-(N + 1) // NS)
    RPT = ((RPT + 127) // 128) * 128
    NACC = NS * RPT

    src = edge_index[0]
    dst = edge_index[1]
    pad = EP - E
    pad_src = jnp.arange(pad, dtype=src.dtype) % N
    pad_dst = N + jnp.arange(pad, dtype=dst.dtype) % (NACC - N)
    srcp = jnp.concatenate([src, pad_src]).reshape(NW, CH, B)
    dstp = jnp.concatenate([dst, pad_dst]).reshape(NW, CH, B)
    srcp = srcp.astype(jnp.int32)
    dstp = dstp.astype(jnp.int32)

    zeros1 = jnp.zeros((RPT,), jnp.float32)
    zerosH = jnp.zeros((RPT, H), jnp.float32)
    zerosC = jnp.zeros((RPT, C2), jnp.float32)

    deg_p = _make_deg_kernel(CH, NACC, RPT)(dstp, zeros1).reshape(NC, NACC)

    xp = jnp.pad(x, ((0, NACC - N), (0, 0)))

    h, g, dis, r = _tc_layer1(xp, W1, deg_p)

    aggi = _make_agg_kernel(N, H, CH, NACC, RPT)(g, srcp, dstp, zerosH)

    W2p = jnp.pad(W2, ((0, 0), (0, C2 - C)))
    b1row = b1[None, :]
    b2row = jnp.pad(b2, (0, C2 - C))[None, :]

    h2, g2 = _tc_layer2(aggi, h, dis, r, b1row, W2p, H)

    agg2i = _make_agg_kernel(N, C2, CH, NACC, RPT)(g2, srcp, dstp, zerosC)

    return _tc_final(agg2i, h2, dis, r, b2row, C, C2)[:N]


# speedup vs baseline: 1.0257x; 1.0012x over previous
import functools

import jax
import jax.numpy as jnp
from jax import lax
from jax.experimental import pallas as pl
from jax.experimental.pallas import tpu as pltpu
from jax.experimental.pallas import tpu_sc as plsc

NC = 2
NS = 16
B = 128

def _mesh():
    return plsc.VectorSubcoreMesh(
        core_axis_name="c", subcore_axis_name="s",
        num_cores=NC, num_subcores=NS,
    )


def _make_deg_kernel(CH, NACC, RPT):
    @functools.partial(
        pl.kernel,
        out_type=jax.ShapeDtypeStruct((NC * NACC,), jnp.float32),
        mesh=_mesh(),
        scratch_types=[
            pltpu.VMEM((CH, B), jnp.int32),
            pltpu.VMEM((B,), jnp.float32),
            pltpu.VMEM_SHARED((NACC,), jnp.float32),
            pltpu.SemaphoreType.DMA,
        ],
        compiler_params=pltpu.CompilerParams(use_tc_tiling_on_sc=False),
    )
    def deg_kernel(dst_hbm, zeros_hbm, out_hbm, dstv, onesv, acc, sem):
        c = lax.axis_index("c")
        s = lax.axis_index("s")
        w = c * NS + s
        pltpu.sync_copy(dst_hbm.at[w], dstv)
        for i in range(B // 16):
            onesv[pl.ds(i * 16, 16)] = jnp.ones((16,), jnp.float32)
        pltpu.sync_copy(zeros_hbm, acc.at[pl.ds(s * RPT, RPT)])
        plsc.subcore_barrier()

        def group(gi, carry):
            for b in range(8):
                pltpu.make_async_copy(
                    onesv, acc.at[dstv.at[8 * gi + b]], sem).start(add=True)
            for b in range(8):
                pltpu.make_async_copy(
                    onesv, acc.at[dstv.at[8 * gi + b]], sem).wait()
            return carry

        lax.fori_loop(0, CH // 8, group, 0)
        plsc.subcore_barrier()
        pltpu.sync_copy(acc.at[pl.ds(s * RPT, RPT)],
                        out_hbm.at[pl.ds(c * NACC + s * RPT, RPT)])

    return deg_kernel


def _make_agg_kernel(N, D, CH, NACC, RPT):
    @functools.partial(
        pl.kernel,
        out_type=jax.ShapeDtypeStruct((NACC, 128), jnp.float32),
        mesh=_mesh(),
        scratch_types=[
            pltpu.VMEM((CH, B), jnp.int32),
            pltpu.VMEM((CH, B), jnp.int32),
            pltpu.VMEM((4, B, D), jnp.float32),
            pltpu.VMEM_SHARED((NACC, D), jnp.float32),
            pltpu.SemaphoreType.DMA,
            pltpu.SemaphoreType.DMA,
            pltpu.SemaphoreType.DMA,
            pltpu.SemaphoreType.DMA,
        ],
        compiler_params=pltpu.CompilerParams(use_tc_tiling_on_sc=False),
    )
    def agg_kernel(g_hbm, src_hbm, dst_hbm, zeros_hbm, out_hbm,
                   srcv, dstv, rows, acc, sem0, sem1, sem2, sem3):
        c = lax.axis_index("c")
        s = lax.axis_index("s")
        w = c * NS + s
        sems = (sem0, sem1, sem2, sem3)
        pltpu.sync_copy(src_hbm.at[w], srcv)
        pltpu.sync_copy(dst_hbm.at[w], dstv)
        pltpu.sync_copy(zeros_hbm, acc.at[pl.ds(s * RPT, RPT)])
        plsc.subcore_barrier()

        def start(ch, b):
            pltpu.async_copy(g_hbm.at[srcv.at[ch]], rows.at[b], sems[b])

        def wait_g(ch, b):
            pltpu.make_async_copy(g_hbm.at[srcv.at[ch]], rows.at[b],
                                  sems[b]).wait()

        start(0, 0)
        start(1, 1)
        start(2, 2)

        def group(gi, carry):
            ch0 = 4 * gi
            for b in range(4):
                ch = ch0 + b

                @pl.when(ch + 3 < CH)
                def _():
                    start(ch + 3, (b + 3) % 4)

                wait_g(ch, b)
                pltpu.sync_copy(rows.at[b], acc.at[dstv.at[ch]], add=True)
            return carry

        lax.fori_loop(0, CH // 4, group, 0)
        plsc.subcore_barrier()
        pltpu.sync_copy(acc.at[pl.ds(s * RPT, RPT)],
                        out_hbm.at[pl.ds(s * RPT, RPT), pl.ds(c * D, D)])

    return agg_kernel


_RB = 1024


def _tc_layer1(xp, W1, degp):
    NP, F = xp.shape
    H = W1.shape[1]

    def body(x_ref, w_ref, d_ref, h_ref, g_ref, dis_ref, r_ref):
        deg = (d_ref[0] + d_ref[1] + 1.0)[:, None]
        dis = lax.rsqrt(deg)
        r = 1.0 / deg
        h = jnp.dot(x_ref[...], w_ref[...], preferred_element_type=jnp.float32)
        h_ref[...] = h
        g_ref[...] = h * dis
        dis_ref[...] = dis[:, 0]
        r_ref[...] = r[:, 0]

    return pl.pallas_call(
        body,
        grid=(NP // _RB,),
        in_specs=[
            pl.BlockSpec((_RB, F), lambda i: (i, 0)),
            pl.BlockSpec((F, H), lambda i: (0, 0)),
            pl.BlockSpec((NC, _RB), lambda i: (0, i)),
        ],
        out_specs=[
            pl.BlockSpec((_RB, H), lambda i: (i, 0)),
            pl.BlockSpec((_RB, H), lambda i: (i, 0)),
            pl.BlockSpec((_RB,), lambda i: (i,)),
            pl.BlockSpec((_RB,), lambda i: (i,)),
        ],
        out_shape=[
            jax.ShapeDtypeStruct((NP, H), jnp.float32),
            jax.ShapeDtypeStruct((NP, H), jnp.float32),
            jax.ShapeDtypeStruct((NP,), jnp.float32),
            jax.ShapeDtypeStruct((NP,), jnp.float32),
        ],
    )(xp, W1, degp)


def _tc_layer2(aggi, h, dis, r, b1row, W2p, H):
    NP = h.shape[0]
    C2 = W2p.shape[1]

    def body(a_ref, h_ref, dis_ref, r_ref, b_ref, w_ref, h2_ref, g2_ref):
        dis = dis_ref[...][:, None]
        agg = a_ref[:, :H] + a_ref[:, H:2 * H]
        y1 = jnp.maximum(
            agg * dis + h_ref[...] * r_ref[...][:, None] + b_ref[...], 0.0)
        h2 = jnp.dot(y1, w_ref[...], preferred_element_type=jnp.float32)
        h2_ref[...] = h2
        g2_ref[...] = h2 * dis

    return pl.pallas_call(
        body,
        grid=(NP // _RB,),
        in_specs=[
            pl.BlockSpec((_RB, 128), lambda i: (i, 0)),
            pl.BlockSpec((_RB, H), lambda i: (i, 0)),
            pl.BlockSpec((_RB,), lambda i: (i,)),
            pl.BlockSpec((_RB,), lambda i: (i,)),
            pl.BlockSpec((1, H), lambda i: (0, 0)),
            pl.BlockSpec((H, C2), lambda i: (0, 0)),
        ],
        out_specs=[
            pl.BlockSpec((_RB, C2), lambda i: (i, 0)),
            pl.BlockSpec((_RB, C2), lambda i: (i, 0)),
        ],
        out_shape=[
            jax.ShapeDtypeStruct((NP, C2), jnp.float32),
            jax.ShapeDtypeStruct((NP, C2), jnp.float32),
        ],
    )(aggi, h, dis, r, b1row, W2p)


def _tc_final(agg2i, h2, dis, r, b2row, C, C2):
    NP = h2.shape[0]

    def body(a_ref, h2_ref, dis_ref, r_ref, b_ref, o_ref):
        agg = a_ref[:, :C2] + a_ref[:, C2:2 * C2]
        val = jnp.maximum(
            agg * dis_ref[...][:, None]
            + h2_ref[...] * r_ref[...][:, None] + b_ref[...], 0.0)
        o_ref[...] = val[:, :C]

    return pl.pallas_call(
        body,
        grid=(NP // _RB,),
        in_specs=[
            pl.BlockSpec((_RB, 128), lambda i: (i, 0)),
            pl.BlockSpec((_RB, C2), lambda i: (i, 0)),
            pl.BlockSpec((_RB,), lambda i: (i,)),
            pl.BlockSpec((_RB,), lambda i: (i,)),
            pl.BlockSpec((1, C2), lambda i: (0, 0)),
        ],
        out_specs=pl.BlockSpec((_RB, C), lambda i: (i, 0)),
        out_shape=jax.ShapeDtypeStruct((NP, C), jnp.float32),
    )(agg2i, h2, dis, r, b2row)


def kernel(x, edge_index, W1, b1, W2, b2):
    N, F = x.shape
    E = edge_index.shape[1]
    H = W1.shape[1]
    C = W2.shape[1]
    C2 = ((C + 15) // 16) * 16

    NW = NC * NS
    CH = -(-E // (NW * B))
    CH = ((CH + 7) // 8) * 8
    EP = NW * CH * B
    RPT = -(-(N + 1) // NS)
    RPT = ((RPT + 127) // 128) * 128
    NACC = NS * RPT

    src = edge_index[0]
    dst = edge_index[1]
    pad = EP - E
    pad_src = jnp.arange(pad, dtype=src.dtype) % N
    pad_dst = N + jnp.arange(pad, dtype=dst.dtype) % (NACC - N)
    srcp = jnp.concatenate([src, pad_src]).reshape(NW, CH, B)
    dstp = jnp.concatenate([dst, pad_dst]).reshape(NW, CH, B)
    srcp = srcp.astype(jnp.int32)
    dstp = dstp.astype(jnp.int32)

    zeros1 = jnp.zeros((RPT,), jnp.float32)
    zerosH = jnp.zeros((RPT, H), jnp.float32)
    zerosC = jnp.zeros((RPT, C2), jnp.float32)

    deg_p = _make_deg_kernel(CH, NACC, RPT)(dstp, zeros1).reshape(NC, NACC)

    xp = jnp.pad(x, ((0, NACC - N), (0, 0)))

    h, g, dis, r = _tc_layer1(xp, W1, deg_p)

    aggi = _make_agg_kernel(N, H, CH, NACC, RPT)(g, srcp, dstp, zerosH)

    W2p = jnp.pad(W2, ((0, 0), (0, C2 - C)))
    b1row = b1[None, :]
    b2row = jnp.pad(b2, (0, C2 - C))[None, :]

    h2, g2 = _tc_layer2(aggi, h, dis, r, b1row, W2p, H)

    agg2i = _make_agg_kernel(N, C2, CH, NACC, RPT)(g2, srcp, dstp, zerosC)

    return _tc_final(agg2i, h2, dis, r, b2row, C, C2)[:N]
